# baseline (speedup 1.0000x reference)
import functools

import jax
import jax.numpy as jnp
from jax import lax
from jax.experimental import pallas as pl
from jax.experimental.pallas import tpu as pltpu
from jax.experimental.pallas import tpu_sc as plsc

N = 10000
E = 320000
D = 128

NC = 2
NS = 16
NW = NC * NS

CHUNK = 128
CPT = 80
EPT = CPT * CHUNK
E_PAD = NW * EPT
N_ACC = 10112
ROWS_PT = N_ACC // NS
DEG_PAD = 10240
DEG_PT = DEG_PAD // NS

_mesh = plsc.VectorSubcoreMesh(core_axis_name="c", subcore_axis_name="s")


def _wid():
    return lax.axis_index("c") * NS + lax.axis_index("s")


@functools.partial(
    pl.kernel,
    out_type=jax.ShapeDtypeStruct((NC, DEG_PAD), jnp.float32),
    mesh=_mesh,
    scratch_types=[
        pltpu.VMEM((CPT, CHUNK), jnp.int32),
        pltpu.VMEM((CHUNK,), jnp.float32),
        pltpu.VMEM((DEG_PT,), jnp.float32),
        pltpu.VMEM_SHARED((DEG_PAD,), jnp.float32),
    ],
)
def _deg_kernel(dst_hbm, out_hbm, idx_v, ones_v, zbuf, deg_sh):
    c = lax.axis_index("c")
    s = lax.axis_index("s")
    w = _wid()

    def fill(i, _):
        zbuf[pl.ds(i * 16, 16)] = jnp.zeros((16,), jnp.float32)
        return 0

    lax.fori_loop(0, DEG_PT // 16, fill, 0)

    def fill1(i, _):
        ones_v[pl.ds(i * 16, 16)] = jnp.full((16,), 1.0, jnp.float32)
        return 0

    lax.fori_loop(0, CHUNK // 16, fill1, 0)

    pltpu.sync_copy(zbuf, deg_sh.at[pl.ds(s * DEG_PT, DEG_PT)])
    plsc.subcore_barrier()

    pltpu.sync_copy(dst_hbm.at[w], idx_v)

    def body(j, _):
        pltpu.sync_copy(ones_v, deg_sh.at[idx_v.at[j]], add=True)
        return 0

    lax.fori_loop(0, CPT, body, 0)
    plsc.subcore_barrier()
    pltpu.sync_copy(deg_sh.at[pl.ds(s * DEG_PT, DEG_PT)],
                    out_hbm.at[c, pl.ds(s * DEG_PT, DEG_PT)])


def _mm_body(x_ref, w_ref, h_ref):
    h_ref[...] = jnp.dot(x_ref[...], w_ref[...],
                         preferred_element_type=jnp.float32)


def _scale_body(h_ref, degp_ref, g_ref):
    deg = degp_ref[0, :N] + degp_ref[1, :N] + 1.0
    dinv = lax.rsqrt(deg)
    g_ref[...] = h_ref[...] * dinv[:, None]


PIECE = 32
P0 = 1
P1 = 4
TOT_CHUNKS = NS * PIECE * (P0 + P1)


@functools.partial(
    pl.kernel,
    out_type=jax.ShapeDtypeStruct((NC, N_ACC, D), jnp.float32),
    mesh=_mesh,
    scratch_types=[
        pltpu.VMEM((PIECE * CHUNK,), jnp.int32),
        pltpu.VMEM((PIECE, CHUNK), jnp.int32),
        pltpu.VMEM((CHUNK, D), jnp.float32),
        pltpu.VMEM((CHUNK, D), jnp.float32),
        pltpu.SemaphoreType.DMA,
        pltpu.SemaphoreType.DMA,
        pltpu.VMEM_SHARED((N_ACC, D), jnp.float32),
    ],
)
def _agg_kernel(g_hbm, srcf_hbm, dst2d_hbm, zeros_hbm, out_hbm,
                idxs, idxd, buf_a, buf_b, sem_a, sem_b, acc_sh):
    c = lax.axis_index("c")
    s = lax.axis_index("s")

    pltpu.sync_copy(zeros_hbm.at[pl.ds(s * ROWS_PT, ROWS_PT)],
                    acc_sh.at[pl.ds(s * ROWS_PT, ROWS_PT)])
    plsc.subcore_barrier()

    npieces = jnp.where(c == 1, P0, P1)
    base_chunk = jnp.where(c == 1, s * (P0 * PIECE),
                           NS * P0 * PIECE + s * (P1 * PIECE))

    def piece_body(p, _):
        pbase = base_chunk + p * PIECE
        pltpu.sync_copy(srcf_hbm.at[pl.ds(pbase * CHUNK, PIECE * CHUNK)],
                        idxs)
        pltpu.sync_copy(dst2d_hbm.at[pl.ds(pbase, PIECE)], idxd)

        pltpu.async_copy(g_hbm.at[idxs.at[pl.ds(0, CHUNK)]], buf_a, sem_a)
        pltpu.async_copy(g_hbm.at[idxs.at[pl.ds(CHUNK, CHUNK)]], buf_b, sem_b)

        def body(i, _):
            for b, (buf, sem) in enumerate(((buf_a, sem_a), (buf_b, sem_b))):
                j = 2 * i + b
                pltpu.make_async_copy(
                    g_hbm.at[idxs.at[pl.ds(j * CHUNK, CHUNK)]],
                    buf, sem).wait()
                pltpu.sync_copy(buf, acc_sh.at[idxd.at[j]], add=True)

                @pl.when(j < PIECE - 2)
                def _():
                    pltpu.async_copy(
                        g_hbm.at[idxs.at[pl.ds((j + 2) * CHUNK, CHUNK)]],
                        buf, sem)
            return 0

        lax.fori_loop(0, PIECE // 2, body, 0)
        return 0

    lax.fori_loop(0, npieces, piece_body, 0)
    plsc.subcore_barrier()
    pltpu.sync_copy(acc_sh.at[pl.ds(s * ROWS_PT, ROWS_PT)],
                    out_hbm.at[c, pl.ds(s * ROWS_PT, ROWS_PT)])


def _combine_body(acc_ref, g_ref, degp_ref, b_ref, out_ref):
    deg = degp_ref[0, :N] + degp_ref[1, :N] + 1.0
    dinv = lax.rsqrt(deg)
    tot = acc_ref[0, :N, :] + acc_ref[1, :N, :] + g_ref[...]
    hr = jnp.maximum(tot * dinv[:, None] + b_ref[...][None, :], 0.0)
    out_ref[:N, :] = hr
    out_ref[N:, :] = jnp.zeros((N_ACC - N, D), jnp.float32)


DCH = 32
DCPT = EPT // DCH


@functools.partial(
    pl.kernel,
    out_type=jax.ShapeDtypeStruct((NW, DCPT, DCH * 16), jnp.float32),
    mesh=_mesh,
    scratch_types=[
        pltpu.VMEM((20 * 2 * DCH,), jnp.int32),
        pltpu.VMEM((2 * DCH, D), jnp.float32),
        pltpu.VMEM((2 * DCH, D), jnp.float32),
        pltpu.VMEM((DCH * 16,), jnp.float32),
        pltpu.VMEM((DCH * 16,), jnp.float32),
        pltpu.SemaphoreType.DMA,
        pltpu.SemaphoreType.DMA,
        pltpu.SemaphoreType.DMA,
        pltpu.SemaphoreType.DMA,
        pltpu.VMEM_SHARED((N_ACC, D), jnp.float32),
    ],
)
def _decode_kernel(hr_hbm, comb_hbm, out_hbm,
                   idx_v, b0, b1, o0, o1, s0, s1, t0, t1, tab_sh):
    s = lax.axis_index("s")
    w = _wid()
    bufs = ((b0, s0), (b1, s1))
    outs = ((o0, t0), (o1, t1))
    piece = 20

    pltpu.sync_copy(hr_hbm.at[pl.ds(s * ROWS_PT, ROWS_PT)],
                    tab_sh.at[pl.ds(s * ROWS_PT, ROWS_PT)])
    plsc.subcore_barrier()

    for h in range(DCPT // 20):
        pltpu.sync_copy(comb_hbm.at[w, pl.ds(h * (20 * 2 * DCH), 20 * 2 * DCH)],
                        idx_v)
        for r in range(2):
            pltpu.async_copy(
                tab_sh.at[idx_v.at[pl.ds(r * 2 * DCH, 2 * DCH)]],
                bufs[r][0], bufs[r][1])

        def body(i, _):
            for r in range(2):
                jj = 2 * i + r
                j = h * piece + jj
                buf, sem = bufs[r]
                outb, osem = outs[r]
                pltpu.make_async_copy(
                    tab_sh.at[idx_v.at[pl.ds(jj * 2 * DCH, 2 * DCH)]],
                    buf, sem).wait()

                if h == 0:
                    @pl.when(jj >= 2)
                    def _():
                        pltpu.make_async_copy(outb, out_hbm.at[w, j],
                                              osem).wait()
                else:
                    pltpu.make_async_copy(outb, out_hbm.at[w, j], osem).wait()

                def edge(e, _):
                    acc = buf[e, pl.ds(0, 16)] * buf[DCH + e, pl.ds(0, 16)]
                    for q in range(1, D // 16):
                        acc = acc + (buf[e, pl.ds(q * 16, 16)]
                                     * buf[DCH + e, pl.ds(q * 16, 16)])
                    outb[pl.ds(16 * e, 16)] = acc
                    return 0

                lax.fori_loop(0, DCH, edge, 0, unroll=2)
                pltpu.async_copy(outb, out_hbm.at[w, j], osem)

                @pl.when(jj + 2 < piece)
                def _():
                    pltpu.async_copy(
                        tab_sh.at[idx_v.at[pl.ds((jj + 2) * 2 * DCH,
                                                 2 * DCH)]],
                        buf, sem)
            return 0

        lax.fori_loop(0, piece // 2, body, 0)

    for r in range(2):
        j = DCPT - 2 + r
        pltpu.make_async_copy(outs[r][0], out_hbm.at[w, j], outs[r][1]).wait()


def _sig_body(p_ref, out_ref):
    r = lax.broadcasted_iota(jnp.int32, (DCH * 16, DCH), 0)
    c = lax.broadcasted_iota(jnp.int32, (DCH * 16, DCH), 1)
    m = jnp.where(r // 16 == c, 1.0, 0.0).astype(jnp.float32)
    s = jnp.dot(p_ref[0], m, preferred_element_type=jnp.float32)
    out_ref[0] = 1.0 / (1.0 + jnp.exp(-s))


def kernel(x, edge_index, W1, b1):
    src = edge_index[0]
    dst = edge_index[1]
    i32 = jnp.int32
    pad = E_PAD - E

    src_flat = jnp.concatenate([src, jnp.zeros((pad,), i32)])
    dst_enc = jnp.concatenate([dst, jnp.full((pad,), N, i32)])
    dst2d = dst_enc.reshape(E_PAD // CHUNK, CHUNK)
    dst_deg = dst_enc.reshape(NW, CPT, CHUNK)
    dst_dec = jnp.concatenate([dst, jnp.zeros((pad,), i32)])
    comb = jnp.concatenate(
        [src_flat.reshape(NW, DCPT, DCH), dst_dec.reshape(NW, DCPT, DCH)],
        axis=2).reshape(NW, 2 * EPT)

    h = pl.pallas_call(
        _mm_body,
        out_shape=jax.ShapeDtypeStruct((N, D), jnp.float32),
    )(x, W1)
    deg_p = _deg_kernel(dst_deg)

    g = pl.pallas_call(
        _scale_body,
        out_shape=jax.ShapeDtypeStruct((N, D), jnp.float32),
    )(h, deg_p)

    zeros = jnp.zeros((N_ACC, D), jnp.float32)
    acc_p = _agg_kernel(g, src_flat, dst2d, zeros)

    hrp = pl.pallas_call(
        _combine_body,
        out_shape=jax.ShapeDtypeStruct((N_ACC, D), jnp.float32),
    )(acc_p, g, deg_p, b1)

    partials = _decode_kernel(hrp, comb)

    sig = pl.pallas_call(
        _sig_body,
        grid=(NW,),
        in_specs=[pl.BlockSpec((1, DCPT, DCH * 16), lambda w: (w, 0, 0))],
        out_specs=pl.BlockSpec((1, DCPT, DCH), lambda w: (w, 0, 0)),
        out_shape=jax.ShapeDtypeStruct((NW, DCPT, DCH), jnp.float32),
    )(partials)
    return sig.reshape(E_PAD)[:E]


# speedup vs baseline: 1.1909x; 1.1909x over previous
import functools

import jax
import jax.numpy as jnp
from jax import lax
from jax.experimental import pallas as pl
from jax.experimental.pallas import tpu as pltpu
from jax.experimental.pallas import tpu_sc as plsc

N = 10000
E = 320000
D = 128

NC = 2
NS = 16
NW = NC * NS

CHUNK = 128
CPT = 80
EPT = CPT * CHUNK
E_PAD = NW * EPT
N_ACC = 10112
ROWS_PT = N_ACC // NS
DEG_PAD = 10240
DEG_PT = DEG_PAD // NS

_mesh = plsc.VectorSubcoreMesh(core_axis_name="c", subcore_axis_name="s")


def _wid():
    return lax.axis_index("c") * NS + lax.axis_index("s")


@functools.partial(
    pl.kernel,
    out_type=jax.ShapeDtypeStruct((NC, DEG_PAD), jnp.float32),
    mesh=_mesh,
    scratch_types=[
        pltpu.VMEM((CPT, CHUNK), jnp.int32),
        pltpu.VMEM((CHUNK,), jnp.float32),
        pltpu.VMEM((DEG_PT,), jnp.float32),
        pltpu.VMEM_SHARED((DEG_PAD,), jnp.float32),
    ],
)
def _deg_kernel(dst_hbm, out_hbm, idx_v, ones_v, zbuf, deg_sh):
    c = lax.axis_index("c")
    s = lax.axis_index("s")
    w = _wid()

    def fill(i, _):
        zbuf[pl.ds(i * 16, 16)] = jnp.zeros((16,), jnp.float32)
        return 0

    lax.fori_loop(0, DEG_PT // 16, fill, 0)

    def fill1(i, _):
        ones_v[pl.ds(i * 16, 16)] = jnp.full((16,), 1.0, jnp.float32)
        return 0

    lax.fori_loop(0, CHUNK // 16, fill1, 0)

    pltpu.sync_copy(zbuf, deg_sh.at[pl.ds(s * DEG_PT, DEG_PT)])
    plsc.subcore_barrier()

    pltpu.sync_copy(dst_hbm.at[w], idx_v)

    def body(j, _):
        pltpu.sync_copy(ones_v, deg_sh.at[idx_v.at[j]], add=True)
        return 0

    lax.fori_loop(0, CPT, body, 0)
    plsc.subcore_barrier()
    pltpu.sync_copy(deg_sh.at[pl.ds(s * DEG_PT, DEG_PT)],
                    out_hbm.at[c, pl.ds(s * DEG_PT, DEG_PT)])


def _mm_body(x_ref, w_ref, degp_ref, g_ref):
    deg = degp_ref[0, :N] + degp_ref[1, :N] + 1.0
    dinv = lax.rsqrt(deg)
    h = jnp.dot(x_ref[...], w_ref[...], preferred_element_type=jnp.float32)
    g_ref[...] = h * dinv[:, None]


PIECE = 32
P0 = 1
P1 = 4
TOT_CHUNKS = NS * PIECE * (P0 + P1)


@functools.partial(
    pl.kernel,
    out_type=jax.ShapeDtypeStruct((NC, N_ACC, D), jnp.float32),
    mesh=_mesh,
    scratch_types=[
        pltpu.VMEM((PIECE * CHUNK,), jnp.int32),
        pltpu.VMEM((PIECE, CHUNK), jnp.int32),
        pltpu.VMEM((CHUNK, D), jnp.float32),
        pltpu.VMEM((CHUNK, D), jnp.float32),
        pltpu.SemaphoreType.DMA,
        pltpu.SemaphoreType.DMA,
        pltpu.VMEM_SHARED((N_ACC, D), jnp.float32),
    ],
)
def _agg_kernel(g_hbm, srcf_hbm, dst2d_hbm, zeros_hbm, out_hbm,
                idxs, idxd, buf_a, buf_b, sem_a, sem_b, acc_sh):
    c = lax.axis_index("c")
    s = lax.axis_index("s")

    pltpu.sync_copy(zeros_hbm.at[pl.ds(s * ROWS_PT, ROWS_PT)],
                    acc_sh.at[pl.ds(s * ROWS_PT, ROWS_PT)])
    plsc.subcore_barrier()

    npieces = jnp.where(c == 1, P0, P1)
    base_chunk = jnp.where(c == 1, s * (P0 * PIECE),
                           NS * P0 * PIECE + s * (P1 * PIECE))

    def piece_body(p, _):
        pbase = base_chunk + p * PIECE
        pltpu.sync_copy(srcf_hbm.at[pl.ds(pbase * CHUNK, PIECE * CHUNK)],
                        idxs)
        pltpu.sync_copy(dst2d_hbm.at[pl.ds(pbase, PIECE)], idxd)

        pltpu.async_copy(g_hbm.at[idxs.at[pl.ds(0, CHUNK)]], buf_a, sem_a)
        pltpu.async_copy(g_hbm.at[idxs.at[pl.ds(CHUNK, CHUNK)]], buf_b, sem_b)

        def body(i, _):
            for b, (buf, sem) in enumerate(((buf_a, sem_a), (buf_b, sem_b))):
                j = 2 * i + b
                pltpu.make_async_copy(
                    g_hbm.at[idxs.at[pl.ds(j * CHUNK, CHUNK)]],
                    buf, sem).wait()
                pltpu.sync_copy(buf, acc_sh.at[idxd.at[j]], add=True)

                @pl.when(j < PIECE - 2)
                def _():
                    pltpu.async_copy(
                        g_hbm.at[idxs.at[pl.ds((j + 2) * CHUNK, CHUNK)]],
                        buf, sem)
            return 0

        lax.fori_loop(0, PIECE // 2, body, 0)
        return 0

    lax.fori_loop(0, npieces, piece_body, 0)
    plsc.subcore_barrier()
    pltpu.sync_copy(acc_sh.at[pl.ds(s * ROWS_PT, ROWS_PT)],
                    out_hbm.at[c, pl.ds(s * ROWS_PT, ROWS_PT)])


def _combine_body(acc_ref, g_ref, degp_ref, b_ref, out_ref):
    deg = degp_ref[0, :N] + degp_ref[1, :N] + 1.0
    dinv = lax.rsqrt(deg)
    tot = acc_ref[0, :N, :] + acc_ref[1, :N, :] + g_ref[...]
    hr = jnp.maximum(tot * dinv[:, None] + b_ref[...][None, :], 0.0)
    out_ref[:N, :] = hr
    out_ref[N:, :] = jnp.zeros((N_ACC - N, D), jnp.float32)


DCH = 32
DCPT = EPT // DCH


@functools.partial(
    pl.kernel,
    out_type=jax.ShapeDtypeStruct((NW, DCPT, DCH * 16), jnp.float32),
    mesh=_mesh,
    scratch_types=[
        pltpu.VMEM((20 * 2 * DCH,), jnp.int32),
        pltpu.VMEM((2 * DCH, D), jnp.float32),
        pltpu.VMEM((2 * DCH, D), jnp.float32),
        pltpu.VMEM((DCH * 16,), jnp.float32),
        pltpu.VMEM((DCH * 16,), jnp.float32),
        pltpu.SemaphoreType.DMA,
        pltpu.SemaphoreType.DMA,
        pltpu.SemaphoreType.DMA,
        pltpu.SemaphoreType.DMA,
        pltpu.VMEM_SHARED((N_ACC, D), jnp.float32),
    ],
)
def _decode_kernel(hr_hbm, comb_hbm, out_hbm,
                   idx_v, b0, b1, o0, o1, s0, s1, t0, t1, tab_sh):
    s = lax.axis_index("s")
    w = _wid()
    bufs = ((b0, s0), (b1, s1))
    outs = ((o0, t0), (o1, t1))
    piece = 20

    pltpu.sync_copy(hr_hbm.at[pl.ds(s * ROWS_PT, ROWS_PT)],
                    tab_sh.at[pl.ds(s * ROWS_PT, ROWS_PT)])
    plsc.subcore_barrier()

    for h in range(DCPT // 20):
        pltpu.sync_copy(comb_hbm.at[w, pl.ds(h * (20 * 2 * DCH), 20 * 2 * DCH)],
                        idx_v)
        for r in range(2):
            pltpu.async_copy(
                tab_sh.at[idx_v.at[pl.ds(r * 2 * DCH, 2 * DCH)]],
                bufs[r][0], bufs[r][1])

        def body(i, _):
            for r in range(2):
                jj = 2 * i + r
                j = h * piece + jj
                buf, sem = bufs[r]
                outb, osem = outs[r]
                pltpu.make_async_copy(
                    tab_sh.at[idx_v.at[pl.ds(jj * 2 * DCH, 2 * DCH)]],
                    buf, sem).wait()

                if h == 0:
                    @pl.when(jj >= 2)
                    def _():
                        pltpu.make_async_copy(outb, out_hbm.at[w, j],
                                              osem).wait()
                else:
                    pltpu.make_async_copy(outb, out_hbm.at[w, j], osem).wait()

                def edge(e, _):
                    acc = buf[e, pl.ds(0, 16)] * buf[DCH + e, pl.ds(0, 16)]
                    for q in range(1, D // 16):
                        acc = acc + (buf[e, pl.ds(q * 16, 16)]
                                     * buf[DCH + e, pl.ds(q * 16, 16)])
                    outb[pl.ds(16 * e, 16)] = acc
                    return 0

                lax.fori_loop(0, DCH, edge, 0, unroll=2)
                pltpu.async_copy(outb, out_hbm.at[w, j], osem)

                @pl.when(jj + 2 < piece)
                def _():
                    pltpu.async_copy(
                        tab_sh.at[idx_v.at[pl.ds((jj + 2) * 2 * DCH,
                                                 2 * DCH)]],
                        buf, sem)
            return 0

        lax.fori_loop(0, piece // 2, body, 0)

    for r in range(2):
        j = DCPT - 2 + r
        pltpu.make_async_copy(outs[r][0], out_hbm.at[w, j], outs[r][1]).wait()


def _sig_body(p_ref, out_ref):
    r = lax.broadcasted_iota(jnp.int32, (DCH * 16, DCH), 0)
    c = lax.broadcasted_iota(jnp.int32, (DCH * 16, DCH), 1)
    m = jnp.where(r // 16 == c, 1.0, 0.0).astype(jnp.float32)
    s = jnp.dot(p_ref[0], m, preferred_element_type=jnp.float32)
    out_ref[0] = 1.0 / (1.0 + jnp.exp(-s))


def kernel(x, edge_index, W1, b1):
    src = edge_index[0]
    dst = edge_index[1]
    i32 = jnp.int32
    pad = E_PAD - E

    src_flat = jnp.concatenate([src, jnp.zeros((pad,), i32)])
    dst_enc = jnp.concatenate([dst, jnp.full((pad,), N, i32)])
    dst2d = dst_enc.reshape(E_PAD // CHUNK, CHUNK)
    dst_deg = dst_enc.reshape(NW, CPT, CHUNK)
    dst_dec = jnp.concatenate([dst, jnp.zeros((pad,), i32)])
    comb = jnp.concatenate(
        [src_flat.reshape(NW, DCPT, DCH), dst_dec.reshape(NW, DCPT, DCH)],
        axis=2).reshape(NW, 2 * EPT)

    deg_p = _deg_kernel(dst_deg)

    g = pl.pallas_call(
        _mm_body,
        out_shape=jax.ShapeDtypeStruct((N, D), jnp.float32),
    )(x, W1, deg_p)

    zeros = jnp.zeros((N_ACC, D), jnp.float32)
    acc_p = _agg_kernel(g, src_flat, dst2d, zeros)

    hrp = pl.pallas_call(
        _combine_body,
        out_shape=jax.ShapeDtypeStruct((N_ACC, D), jnp.float32),
    )(acc_p, g, deg_p, b1)

    partials = _decode_kernel(hrp, comb)

    sig = pl.pallas_call(
        _sig_body,
        grid=(NW,),
        in_specs=[pl.BlockSpec((1, DCPT, DCH * 16), lambda w: (w, 0, 0))],
        out_specs=pl.BlockSpec((1, DCPT, DCH), lambda w: (w, 0, 0)),
        out_shape=jax.ShapeDtypeStruct((NW, DCPT, DCH), jnp.float32),
    )(partials)
    return sig.reshape(E_PAD)[:E]
